# Initial kernel scaffold; baseline (speedup 1.0000x reference)
import jax
import jax.numpy as jnp
from jax.experimental import pallas as pl


def kernel(input, w_in, b_in, w_out, b_out, dw_w, dw_b, ln_g, ln_b, w_off, b_off, w_mask, b_mask):
    raise NotImplementedError("write your pallas kernel here")


# speedup vs baseline: 219.9214x; 219.9214x over previous
import functools

import jax
import jax.numpy as jnp
from jax import lax
from jax.experimental import pallas as pl
from jax.experimental.pallas import tpu as pltpu
from jax.experimental.pallas import tpu_sc as plsc

N = 2
H = 64
W = 64
C = 384
G = 24
GC = C // G
P = 9
HW = H * W

NC = 2
NS = 16
NWORK = NC * NS
HALVES = 2
UNITS = N * G * HALVES
UPW = UNITS // NWORK
CHUNK = 512
CHUNKS_PER_HALF = (HW // HALVES) // CHUNK


def _erf(x):
    s = jnp.sign(x)
    a = jnp.abs(x)
    t = 1.0 / (1.0 + 0.3275911 * a)
    y = 1.0 - (((((1.061405429 * t - 1.453152027) * t) + 1.421413741) * t
                - 0.284496736) * t + 0.254829592) * t * jnp.exp(-a * a)
    return s * y


def _gelu(x):
    return 0.5 * x * (1.0 + _erf(x * 0.7071067811865476))


def _conv_body(inp_ref, dw9_ref, dw_b_ref, ln_g_ref, ln_b_ref, x1_ref):
    inp = inp_ref[0]
    li = lax.broadcasted_iota(jnp.int32, (1, HW), 1)
    h_l = li >> 6
    w_l = li & 63
    x1_ref[0] = dw_b_ref[...] * jnp.ones((C, HW), jnp.float32)
    k = 0
    for kh in range(3):
        for kw in range(3):
            dy = kh - 1
            dx = kw - 1
            sh = -(dy * W + dx)
            rolled = jnp.roll(inp, sh, axis=1) if sh != 0 else inp
            cond = jnp.full((1, HW), True)
            if dy == -1:
                cond = cond & (h_l >= 1)
            if dy == 1:
                cond = cond & (h_l <= H - 2)
            if dx == -1:
                cond = cond & (w_l >= 1)
            if dx == 1:
                cond = cond & (w_l <= W - 2)
            rolled = jnp.where(cond, rolled, 0.0)
            x1_ref[0] = x1_ref[0] + dw9_ref[:, k:k + 1] * rolled
            k += 1

    x1 = x1_ref[0]
    m = jnp.mean(x1, axis=0, keepdims=True)
    xc = x1 - m
    v = jnp.mean(xc * xc, axis=0, keepdims=True)
    x1 = xc * lax.rsqrt(v + 1e-6) * ln_g_ref[...] + ln_b_ref[...]
    x1_ref[0] = _gelu(x1)


def _proj_body(inp_ref, x1_ref, w_in_ref, b_in_ref, w_off_ref, b_off_ref,
               w_mask_ref, b_mask_ref, xT_ref, offT_ref, mlogT_ref):
    xT_ref[0] = jnp.dot(w_in_ref[...], inp_ref[0],
                        preferred_element_type=jnp.float32) + b_in_ref[...]
    x1 = x1_ref[0]
    offT_ref[0] = jnp.dot(w_off_ref[...], x1,
                          preferred_element_type=jnp.float32) + b_off_ref[...]
    mlogT_ref[0] = jnp.dot(w_mask_ref[...], x1,
                           preferred_element_type=jnp.float32) + b_mask_ref[...]


def _post_body(y_ref, w_out_ref, b_out_ref, out_ref):
    out_ref[0] = jnp.dot(w_out_ref[...], y_ref[0],
                         preferred_element_type=jnp.float32) + b_out_ref[...]


def _sc_body(xT, offT, mlogT, yT, x_v, off_v, msk_v, out_v):
    wid = lax.axis_index("s") * NC + lax.axis_index("c")
    iota16 = jnp.arange(16, dtype=jnp.int32)

    def unit_body(u, _):
        unit = wid * UPW + u
        n = unit // (G * HALVES)
        rem = unit % (G * HALVES)
        g = rem // HALVES
        half = rem % HALVES

        pltpu.sync_copy(xT.at[n, g], x_v)

        def chunk_body(chunk, _):
            base = half * (HW // HALVES) + chunk * CHUNK
            pltpu.sync_copy(offT.at[n, g, :, pl.ds(base, CHUNK)], off_v)
            pltpu.sync_copy(mlogT.at[n, g, :, pl.ds(base, CHUNK)], msk_v)

            def sub_body(s, _):
                o16 = s * 16
                pvec = base + o16 + iota16
                hh = (pvec >> 6).astype(jnp.float32)
                ww = (pvec & 63).astype(jnp.float32)

                logits = [msk_v[p, pl.ds(o16, 16)] for p in range(P)]
                mx = logits[0]
                for p in range(1, P):
                    mx = jnp.maximum(mx, logits[p])
                exps = [jnp.exp(l - mx) for l in logits]
                ssum = exps[0]
                for p in range(1, P):
                    ssum = ssum + exps[p]
                inv = 1.0 / ssum

                accs = [jnp.zeros((16,), jnp.float32) for _ in range(GC)]
                for p in range(P):
                    dx = p // 3 - 1
                    dy = p % 3 - 1
                    fx = ww + (off_v[2 * p, pl.ds(o16, 16)] + float(dx))
                    fy = hh + (off_v[2 * p + 1, pl.ds(o16, 16)] + float(dy))
                    xi = fx.astype(jnp.int32)
                    xi = jnp.where(xi.astype(jnp.float32) > fx, xi - 1, xi)
                    yi = fy.astype(jnp.int32)
                    yi = jnp.where(yi.astype(jnp.float32) > fy, yi - 1, yi)
                    wx1 = fx - xi.astype(jnp.float32)
                    wx0 = 1.0 - wx1
                    wy1 = fy - yi.astype(jnp.float32)
                    wy0 = 1.0 - wy1
                    vx0 = jnp.where((xi >= 0) & (xi <= W - 1), 1.0, 0.0)
                    vx1 = jnp.where((xi >= -1) & (xi <= W - 2), 1.0, 0.0)
                    vy0 = jnp.where((yi >= 0) & (yi <= H - 1), 1.0, 0.0)
                    vy1 = jnp.where((yi >= -1) & (yi <= H - 2), 1.0, 0.0)
                    xc0 = jnp.clip(xi, 0, W - 1)
                    xc1 = jnp.clip(xi + 1, 0, W - 1)
                    yc0 = jnp.clip(yi, 0, H - 1)
                    yc1 = jnp.clip(yi + 1, 0, H - 1)
                    b00 = yc0 * W + xc0
                    b01 = yc0 * W + xc1
                    b10 = yc1 * W + xc0
                    b11 = yc1 * W + xc1
                    mp = exps[p] * inv
                    t00 = mp * (wx0 * wy0 * (vx0 * vy0))
                    t01 = mp * (wx1 * wy0 * (vx1 * vy0))
                    t10 = mp * (wx0 * wy1 * (vx0 * vy1))
                    t11 = mp * (wx1 * wy1 * (vx1 * vy1))
                    for c in range(GC):
                        ci = jnp.full((16,), c, jnp.int32)
                        v00 = plsc.load_gather(x_v, [ci, b00])
                        v01 = plsc.load_gather(x_v, [ci, b01])
                        v10 = plsc.load_gather(x_v, [ci, b10])
                        v11 = plsc.load_gather(x_v, [ci, b11])
                        accs[c] = accs[c] + (t00 * v00 + t01 * v01
                                             + t10 * v10 + t11 * v11)
                for c in range(GC):
                    out_v[c, pl.ds(o16, 16)] = accs[c]
                return 0

            lax.fori_loop(0, CHUNK // 16, sub_body, 0)
            pltpu.sync_copy(out_v, yT.at[n, g, :, pl.ds(base, CHUNK)])
            return 0

        lax.fori_loop(0, CHUNKS_PER_HALF, chunk_body, 0)
        return 0

    lax.fori_loop(0, UPW, unit_body, 0)


@jax.jit
def _run(inpT, w_in, b_in2, dw9, dw_b2, ln_g2, ln_b2, w_off, b_off2,
         w_mask, b_mask2, w_out, b_out2):
    f32 = jnp.float32
    GP2 = G * P * 2
    GP = G * P
    x1 = pl.pallas_call(
        _conv_body,
        grid=(N,),
        in_specs=[
            pl.BlockSpec((1, C, HW), lambda n: (n, 0, 0)),
            pl.BlockSpec((C, P), lambda n: (0, 0)),
            pl.BlockSpec((C, 1), lambda n: (0, 0)),
            pl.BlockSpec((C, 1), lambda n: (0, 0)),
            pl.BlockSpec((C, 1), lambda n: (0, 0)),
        ],
        out_specs=pl.BlockSpec((1, C, HW), lambda n: (n, 0, 0)),
        out_shape=jax.ShapeDtypeStruct((N, C, HW), f32),
    )(inpT, dw9, dw_b2, ln_g2, ln_b2)

    T = 4
    TL = HW // T
    xT, offT, mlogT = pl.pallas_call(
        _proj_body,
        grid=(N, T),
        in_specs=[
            pl.BlockSpec((1, C, TL), lambda n, t: (n, 0, t)),
            pl.BlockSpec((1, C, TL), lambda n, t: (n, 0, t)),
            pl.BlockSpec((C, C), lambda n, t: (0, 0)),
            pl.BlockSpec((C, 1), lambda n, t: (0, 0)),
            pl.BlockSpec((GP2, C), lambda n, t: (0, 0)),
            pl.BlockSpec((GP2, 1), lambda n, t: (0, 0)),
            pl.BlockSpec((GP, C), lambda n, t: (0, 0)),
            pl.BlockSpec((GP, 1), lambda n, t: (0, 0)),
        ],
        out_specs=[
            pl.BlockSpec((1, C, TL), lambda n, t: (n, 0, t)),
            pl.BlockSpec((1, GP2, TL), lambda n, t: (n, 0, t)),
            pl.BlockSpec((1, GP, TL), lambda n, t: (n, 0, t)),
        ],
        out_shape=[
            jax.ShapeDtypeStruct((N, C, HW), f32),
            jax.ShapeDtypeStruct((N, GP2, HW), f32),
            jax.ShapeDtypeStruct((N, GP, HW), f32),
        ],
    )(inpT, x1, w_in, b_in2, w_off, b_off2, w_mask, b_mask2)

    xT_s = xT.reshape(N, G, GC, HW)
    offT_s = offT.reshape(N, G, 2 * P, HW)
    mlogT_s = mlogT.reshape(N, G, P, HW)

    sc = functools.partial(
        pl.kernel,
        out_type=jax.ShapeDtypeStruct((N, G, GC, HW), f32),
        mesh=plsc.VectorSubcoreMesh(core_axis_name="c", subcore_axis_name="s"),
        compiler_params=pltpu.CompilerParams(needs_layout_passes=False),
        scratch_types=[
            pltpu.VMEM((GC, HW), f32),
            pltpu.VMEM((2 * P, CHUNK), f32),
            pltpu.VMEM((P, CHUNK), f32),
            pltpu.VMEM((GC, CHUNK), f32),
        ],
    )(_sc_body)
    yT = sc(xT_s, offT_s, mlogT_s)

    outT = pl.pallas_call(
        _post_body,
        grid=(N,),
        in_specs=[
            pl.BlockSpec((1, C, HW), lambda n: (n, 0, 0)),
            pl.BlockSpec((C, C), lambda n: (0, 0)),
            pl.BlockSpec((C, 1), lambda n: (0, 0)),
        ],
        out_specs=pl.BlockSpec((1, C, HW), lambda n: (n, 0, 0)),
        out_shape=jax.ShapeDtypeStruct((N, C, HW), f32),
    )(yT.reshape(N, C, HW), w_out, b_out2)
    return outT


def kernel(input, w_in, b_in, w_out, b_out, dw_w, dw_b, ln_g, ln_b,
           w_off, b_off, w_mask, b_mask):
    inpT = jnp.transpose(input.reshape(N, HW, C), (0, 2, 1))
    outT = _run(inpT, w_in, b_in[:, None], dw_w.reshape(C, P),
                dw_b[:, None], ln_g[:, None], ln_b[:, None],
                w_off, b_off[:, None], w_mask, b_mask[:, None],
                w_out, b_out[:, None])
    return jnp.transpose(outT, (0, 2, 1)).reshape(N, H, W, C)


# speedup vs baseline: 238.4927x; 1.0844x over previous
import functools

import jax
import jax.numpy as jnp
from jax import lax
from jax.experimental import pallas as pl
from jax.experimental.pallas import tpu as pltpu
from jax.experimental.pallas import tpu_sc as plsc

N = 2
H = 64
W = 64
C = 384
G = 24
GC = C // G
P = 9
HW = H * W

NC = 2
NS = 16
NWORK = NC * NS
HALVES = 2
UNITS = N * G * HALVES
UPW = UNITS // NWORK
CHUNK = 512
CHUNKS_PER_HALF = (HW // HALVES) // CHUNK


def _erf(x):
    s = jnp.sign(x)
    a = jnp.abs(x)
    t = 1.0 / (1.0 + 0.3275911 * a)
    y = 1.0 - (((((1.061405429 * t - 1.453152027) * t) + 1.421413741) * t
                - 0.284496736) * t + 0.254829592) * t * jnp.exp(-a * a)
    return s * y


def _gelu(x):
    return 0.5 * x * (1.0 + _erf(x * 0.7071067811865476))


def _conv_body(inp_ref, dw9_ref, dw_b_ref, ln_g_ref, ln_b_ref, x1_ref):
    inp = inp_ref[0]
    li = lax.broadcasted_iota(jnp.int32, (1, HW), 1)
    h_l = li >> 6
    w_l = li & 63
    x1_ref[0] = dw_b_ref[...] * jnp.ones((C, HW), jnp.float32)
    k = 0
    for kh in range(3):
        for kw in range(3):
            dy = kh - 1
            dx = kw - 1
            sh = -(dy * W + dx)
            rolled = jnp.roll(inp, sh, axis=1) if sh != 0 else inp
            cond = jnp.full((1, HW), True)
            if dy == -1:
                cond = cond & (h_l >= 1)
            if dy == 1:
                cond = cond & (h_l <= H - 2)
            if dx == -1:
                cond = cond & (w_l >= 1)
            if dx == 1:
                cond = cond & (w_l <= W - 2)
            rolled = jnp.where(cond, rolled, 0.0)
            x1_ref[0] = x1_ref[0] + dw9_ref[:, k:k + 1] * rolled
            k += 1

    x1 = x1_ref[0]
    m = jnp.mean(x1, axis=0, keepdims=True)
    xc = x1 - m
    v = jnp.mean(xc * xc, axis=0, keepdims=True)
    x1 = xc * lax.rsqrt(v + 1e-6) * ln_g_ref[...] + ln_b_ref[...]
    x1_ref[0] = _gelu(x1)


def _proj_body(inp_ref, x1_ref, w_in_ref, b_in_ref, w_off_ref, b_off_ref,
               w_mask_ref, b_mask_ref, xT_ref, offT_ref, mlogT_ref):
    xT_ref[0] = jnp.dot(w_in_ref[...], inp_ref[0],
                        preferred_element_type=jnp.float32) + b_in_ref[...]
    x1 = x1_ref[0]
    offT_ref[0] = jnp.dot(w_off_ref[...], x1,
                          preferred_element_type=jnp.float32) + b_off_ref[...]
    mlogT_ref[0] = jnp.dot(w_mask_ref[...], x1,
                           preferred_element_type=jnp.float32) + b_mask_ref[...]


def _post_body(y_ref, w_out_ref, b_out_ref, out_ref):
    out_ref[0] = jnp.dot(w_out_ref[...], y_ref[0],
                         preferred_element_type=jnp.float32) + b_out_ref[...]


def _sc_body(xT, offT, mlogT, yT, x_v, off_v, msk_v, out_v):
    wid = lax.axis_index("s") * NC + lax.axis_index("c")
    iota16 = jnp.arange(16, dtype=jnp.int32)

    def unit_body(u, _):
        unit = wid * UPW + u
        n = unit // (G * HALVES)
        rem = unit % (G * HALVES)
        g = rem // HALVES
        half = rem % HALVES

        pltpu.sync_copy(xT.at[n, g], x_v)

        def chunk_body(chunk, _):
            base = half * (HW // HALVES) + chunk * CHUNK
            pltpu.sync_copy(offT.at[n, g, :, pl.ds(base, CHUNK)], off_v)
            pltpu.sync_copy(mlogT.at[n, g, :, pl.ds(base, CHUNK)], msk_v)

            def sub_body(s, _):
                o16 = s * 16
                pvec = base + o16 + iota16
                hh = (pvec >> 6).astype(jnp.float32)
                ww = (pvec & 63).astype(jnp.float32)

                logits = [msk_v[p, pl.ds(o16, 16)] for p in range(P)]
                mx = logits[0]
                for p in range(1, P):
                    mx = jnp.maximum(mx, logits[p])
                exps = [jnp.exp(l - mx) for l in logits]
                ssum = exps[0]
                for p in range(1, P):
                    ssum = ssum + exps[p]
                inv = 1.0 / ssum

                accs = [jnp.zeros((16,), jnp.float32) for _ in range(GC)]
                for p in range(P):
                    dx = p // 3 - 1
                    dy = p % 3 - 1
                    fx = ww + (off_v[2 * p, pl.ds(o16, 16)] + float(dx))
                    fy = hh + (off_v[2 * p + 1, pl.ds(o16, 16)] + float(dy))
                    xi = fx.astype(jnp.int32)
                    xi = jnp.where(xi.astype(jnp.float32) > fx, xi - 1, xi)
                    yi = fy.astype(jnp.int32)
                    yi = jnp.where(yi.astype(jnp.float32) > fy, yi - 1, yi)
                    wx1 = fx - xi.astype(jnp.float32)
                    wx0 = 1.0 - wx1
                    wy1 = fy - yi.astype(jnp.float32)
                    wy0 = 1.0 - wy1
                    vx0 = jnp.where((xi >= 0) & (xi <= W - 1), 1.0, 0.0)
                    vx1 = jnp.where((xi >= -1) & (xi <= W - 2), 1.0, 0.0)
                    vy0 = jnp.where((yi >= 0) & (yi <= H - 1), 1.0, 0.0)
                    vy1 = jnp.where((yi >= -1) & (yi <= H - 2), 1.0, 0.0)
                    xc0 = jnp.clip(xi, 0, W - 1)
                    xc1 = jnp.clip(xi + 1, 0, W - 1)
                    yc0 = jnp.clip(yi, 0, H - 1)
                    yc1 = jnp.clip(yi + 1, 0, H - 1)
                    b00 = yc0 * W + xc0
                    b01 = yc0 * W + xc1
                    b10 = yc1 * W + xc0
                    b11 = yc1 * W + xc1
                    mp = exps[p] * inv
                    t00 = mp * (wx0 * wy0 * (vx0 * vy0))
                    t01 = mp * (wx1 * wy0 * (vx1 * vy0))
                    t10 = mp * (wx0 * wy1 * (vx0 * vy1))
                    t11 = mp * (wx1 * wy1 * (vx1 * vy1))
                    for c in range(GC):
                        row = x_v.at[pl.ds(c * HW, HW)]
                        v00 = plsc.load_gather(row, [b00])
                        v01 = plsc.load_gather(row, [b01])
                        v10 = plsc.load_gather(row, [b10])
                        v11 = plsc.load_gather(row, [b11])
                        accs[c] = accs[c] + (t00 * v00 + t01 * v01
                                             + t10 * v10 + t11 * v11)
                for c in range(GC):
                    out_v[c, pl.ds(o16, 16)] = accs[c]
                return 0

            lax.fori_loop(0, CHUNK // 16, sub_body, 0)
            pltpu.sync_copy(out_v, yT.at[n, g, :, pl.ds(base, CHUNK)])
            return 0

        lax.fori_loop(0, CHUNKS_PER_HALF, chunk_body, 0)
        return 0

    lax.fori_loop(0, UPW, unit_body, 0)


@jax.jit
def _run(inpT, w_in, b_in2, dw9, dw_b2, ln_g2, ln_b2, w_off, b_off2,
         w_mask, b_mask2, w_out, b_out2):
    f32 = jnp.float32
    GP2 = G * P * 2
    GP = G * P
    x1 = pl.pallas_call(
        _conv_body,
        grid=(N,),
        in_specs=[
            pl.BlockSpec((1, C, HW), lambda n: (n, 0, 0)),
            pl.BlockSpec((C, P), lambda n: (0, 0)),
            pl.BlockSpec((C, 1), lambda n: (0, 0)),
            pl.BlockSpec((C, 1), lambda n: (0, 0)),
            pl.BlockSpec((C, 1), lambda n: (0, 0)),
        ],
        out_specs=pl.BlockSpec((1, C, HW), lambda n: (n, 0, 0)),
        out_shape=jax.ShapeDtypeStruct((N, C, HW), f32),
    )(inpT, dw9, dw_b2, ln_g2, ln_b2)

    T = 4
    TL = HW // T
    xT, offT, mlogT = pl.pallas_call(
        _proj_body,
        grid=(N, T),
        in_specs=[
            pl.BlockSpec((1, C, TL), lambda n, t: (n, 0, t)),
            pl.BlockSpec((1, C, TL), lambda n, t: (n, 0, t)),
            pl.BlockSpec((C, C), lambda n, t: (0, 0)),
            pl.BlockSpec((C, 1), lambda n, t: (0, 0)),
            pl.BlockSpec((GP2, C), lambda n, t: (0, 0)),
            pl.BlockSpec((GP2, 1), lambda n, t: (0, 0)),
            pl.BlockSpec((GP, C), lambda n, t: (0, 0)),
            pl.BlockSpec((GP, 1), lambda n, t: (0, 0)),
        ],
        out_specs=[
            pl.BlockSpec((1, C, TL), lambda n, t: (n, 0, t)),
            pl.BlockSpec((1, GP2, TL), lambda n, t: (n, 0, t)),
            pl.BlockSpec((1, GP, TL), lambda n, t: (n, 0, t)),
        ],
        out_shape=[
            jax.ShapeDtypeStruct((N, C, HW), f32),
            jax.ShapeDtypeStruct((N, GP2, HW), f32),
            jax.ShapeDtypeStruct((N, GP, HW), f32),
        ],
    )(inpT, x1, w_in, b_in2, w_off, b_off2, w_mask, b_mask2)

    xT_s = xT.reshape(N, G, GC * HW)
    offT_s = offT.reshape(N, G, 2 * P, HW)
    mlogT_s = mlogT.reshape(N, G, P, HW)

    sc = functools.partial(
        pl.kernel,
        out_type=jax.ShapeDtypeStruct((N, G, GC, HW), f32),
        mesh=plsc.VectorSubcoreMesh(core_axis_name="c", subcore_axis_name="s"),
        compiler_params=pltpu.CompilerParams(needs_layout_passes=False),
        scratch_types=[
            pltpu.VMEM((GC * HW,), f32),
            pltpu.VMEM((2 * P, CHUNK), f32),
            pltpu.VMEM((P, CHUNK), f32),
            pltpu.VMEM((GC, CHUNK), f32),
        ],
    )(_sc_body)
    yT = sc(xT_s, offT_s, mlogT_s)

    outT = pl.pallas_call(
        _post_body,
        grid=(N,),
        in_specs=[
            pl.BlockSpec((1, C, HW), lambda n: (n, 0, 0)),
            pl.BlockSpec((C, C), lambda n: (0, 0)),
            pl.BlockSpec((C, 1), lambda n: (0, 0)),
        ],
        out_specs=pl.BlockSpec((1, C, HW), lambda n: (n, 0, 0)),
        out_shape=jax.ShapeDtypeStruct((N, C, HW), f32),
    )(yT.reshape(N, C, HW), w_out, b_out2)
    return outT


def kernel(input, w_in, b_in, w_out, b_out, dw_w, dw_b, ln_g, ln_b,
           w_off, b_off, w_mask, b_mask):
    inpT = jnp.transpose(input.reshape(N, HW, C), (0, 2, 1))
    outT = _run(inpT, w_in, b_in[:, None], dw_w.reshape(C, P),
                dw_b[:, None], ln_g[:, None], ln_b[:, None],
                w_off, b_off[:, None], w_mask, b_mask[:, None],
                w_out, b_out[:, None])
    return jnp.transpose(outT, (0, 2, 1)).reshape(N, H, W, C)
